# baseline (speedup 1.0000x reference)
import functools

import jax
import jax.numpy as jnp
from jax import lax
from jax.experimental import pallas as pl
from jax.experimental.pallas import tpu as pltpu
from jax.experimental.pallas import tpu_sc as plsc

_ROWS = 65536
_D = 256
_BLK = 2048


def _fused_body(y_ref, c_ref, w_ref, out_ref):
    out_ref[...] = w_ref[...]

    @pl.when(pl.program_id(0) == 0)
    def _():
        c17 = c_ref[17]
        c18 = c_ref[18]
        c19 = c_ref[19]
        out_ref[0:1, :] = jnp.full((1, _D), c19 * c17, jnp.float32)
        out_ref[1:2, :] = jnp.full((1, _D), c18 / c19, jnp.float32)
        out_ref[2:3, :] = y_ref[3:4, :] + y_ref[1:2, :] + 2.0 * y_ref[2:3, :]


def _fused(y, w, c):
    grid = (_ROWS // _BLK,)
    return pl.pallas_call(
        _fused_body,
        out_shape=jax.ShapeDtypeStruct((_ROWS, _D), jnp.float32),
        grid=grid,
        in_specs=[
            pl.BlockSpec((8, _D), lambda i: (0, 0)),
            pl.BlockSpec(memory_space=pltpu.SMEM),
            pl.BlockSpec((_BLK, _D), lambda i: (i, 0)),
        ],
        out_specs=pl.BlockSpec((_BLK, _D), lambda i: (i, 0)),
        compiler_params=pltpu.CompilerParams(
            dimension_semantics=("arbitrary",),
        ),
    )(y, c, w)


def kernel(y, w, c, t):
    del t
    return _fused(y, w, c)


# speedup vs baseline: 1.1409x; 1.1409x over previous
import functools

import jax
import jax.numpy as jnp
from jax import lax
from jax.experimental import pallas as pl
from jax.experimental.pallas import tpu as pltpu
from jax.experimental.pallas import tpu_sc as plsc

_ROWS = 65536
_D = 256
_BLK = 16352


def _fused_body(y_ref, c_ref, w_ref, out_ref):
    out_ref[...] = w_ref[...]

    @pl.when(pl.program_id(0) == 0)
    def _():
        c17 = c_ref[17]
        c18 = c_ref[18]
        c19 = c_ref[19]
        out_ref[0:1, :] = jnp.full((1, _D), c19 * c17, jnp.float32)
        out_ref[1:2, :] = jnp.full((1, _D), c18 / c19, jnp.float32)
        out_ref[2:3, :] = y_ref[3:4, :] + y_ref[1:2, :] + 2.0 * y_ref[2:3, :]


def _fused(y, w, c):
    grid = (_ROWS // _BLK,)
    return pl.pallas_call(
        _fused_body,
        out_shape=jax.ShapeDtypeStruct((_ROWS, _D), jnp.float32),
        grid=grid,
        in_specs=[
            pl.BlockSpec((8, _D), lambda i: (0, 0)),
            pl.BlockSpec(memory_space=pltpu.SMEM),
            pl.BlockSpec((_BLK, _D), lambda i: (i, 0)),
        ],
        out_specs=pl.BlockSpec((_BLK, _D), lambda i: (i, 0)),
        compiler_params=pltpu.CompilerParams(
            dimension_semantics=("arbitrary",),
            vmem_limit_bytes=134217728,
        ),
    )(y, c, w)


def kernel(y, w, c, t):
    del t
    return _fused(y, w, c)
